# baseline (speedup 1.0000x reference)
import jax
import jax.numpy as jnp
from jax.experimental import pallas as pl


def _combine_kernel(a_ref, b_ref, o_ref):
    o_ref[...] = a_ref[...] + b_ref[...]


def _combine(a, b):
    a2 = jnp.broadcast_to(a, (8, 128))
    b2 = jnp.broadcast_to(b, (8, 128))
    out = pl.pallas_call(
        _combine_kernel,
        out_shape=jax.ShapeDtypeStruct((8, 128), jnp.float32),
    )(a2, b2)
    return out[0, 0]


def kernel(x, edge_index, batch, n_vertices, gt_vertices, W_in, b_in, W_h, b_h, W_z, b_z, codebooks, proj, query_emb, W_kv, b_kv, W_d1, b_d1, W_d2, b_d2):
    N = x.shape[0]
    Bn = n_vertices.shape[0]
    M = gt_vertices.shape[1]
    d_embed = W_z.shape[1]
    h = jax.nn.relu(x @ W_in + b_in)
    src = edge_index[0]
    dst = edge_index[1]
    msg = jax.ops.segment_sum(h[src], dst, num_segments=N)
    deg = jax.ops.segment_sum(jnp.ones((src.shape[0],), jnp.float32), dst, num_segments=N)
    msg = msg / jnp.maximum(deg, 1.0)[:, None]
    h2 = jax.nn.relu((h + msg) @ W_h + b_h)
    psum = jax.ops.segment_sum(h2, batch, num_segments=Bn)
    pcnt = jax.ops.segment_sum(jnp.ones((N,), jnp.float32), batch, num_segments=Bn)
    pooled = psum / jnp.maximum(pcnt, 1.0)[:, None]
    z = pooled @ W_z + b_z
    r = z
    z_q = jnp.zeros_like(z)
    idx_list = []
    for l in range(codebooks.shape[0]):
        codes = codebooks[l] @ proj[l]
        d2 = jnp.sum(r * r, axis=1, keepdims=True) - 2.0 * (r @ codes.T) + jnp.sum(codes * codes, axis=1)[None, :]
        idx = jnp.argmin(d2, axis=1)
        q = codes[idx]
        z_q = z_q + q
        r = r - q
        idx_list.append(idx)
    indices = jnp.stack(idx_list, axis=1)
    z_q_st = z + jax.lax.stop_gradient(z_q - z)
    nkv = W_kv.shape[1] // d_embed
    kv = (z_q_st @ W_kv + b_kv).reshape(Bn, nkv, d_embed)
    logits = jnp.einsum('md,bkd->bmk', query_emb, kv) / jnp.sqrt(jnp.float32(d_embed))
    att = jax.nn.softmax(logits, axis=-1)
    ctx = jnp.einsum('bmk,bkd->bmd', att, kv)
    recon = jax.nn.relu(ctx @ W_d1 + b_d1) @ W_d2 + b_d2
    mask = (jnp.arange(M)[None, :] < n_vertices[:, None]).astype(jnp.float32)
    diff = recon[:, :, None, :] - gt_vertices[:, None, :, :]
    dist = jnp.sum(diff * diff, axis=-1)
    big = 1e9
    d_rg = dist + (1.0 - mask[:, None, :]) * big
    min_rg = jnp.min(d_rg, axis=2)
    denom = jnp.maximum(jnp.sum(mask, axis=1), 1.0)
    loss1 = jnp.sum(min_rg * mask, axis=1) / denom
    d_gr = dist + (1.0 - mask[:, :, None]) * big
    min_gr = jnp.min(d_gr, axis=1)
    loss2 = jnp.sum(min_gr * mask, axis=1) / denom
    recon_loss = jnp.mean(loss1 + loss2)
    commit_loss = jnp.mean((z - jax.lax.stop_gradient(z_q)) ** 2)
    embed_loss = jnp.mean((jax.lax.stop_gradient(z) - z_q) ** 2)
    total_loss = _combine(recon_loss, commit_loss + embed_loss)
    return (recon, total_loss, recon_loss, commit_loss, embed_loss, indices, z)


# speedup vs baseline: 1.0317x; 1.0317x over previous
import jax
import jax.numpy as jnp
from jax.experimental import pallas as pl


def _combine_kernel(a_ref, b_ref, o_ref):
    o_ref[...] = a_ref[...] + b_ref[...]


def _combine(a, b):
    a2 = jnp.broadcast_to(a, (8, 128))
    b2 = jnp.broadcast_to(b, (8, 128))
    out = pl.pallas_call(
        _combine_kernel,
        out_shape=jax.ShapeDtypeStruct((8, 128), jnp.float32),
    )(a2, b2)
    return out[0, 0]


def _bdot(a, b):
    return jax.lax.dot(a.astype(jnp.bfloat16), b.astype(jnp.bfloat16), preferred_element_type=jnp.float32)


def kernel(x, edge_index, batch, n_vertices, gt_vertices, W_in, b_in, W_h, b_h, W_z, b_z, codebooks, proj, query_emb, W_kv, b_kv, W_d1, b_d1, W_d2, b_d2):
    N = x.shape[0]
    Bn = n_vertices.shape[0]
    M = gt_vertices.shape[1]
    d_embed = W_z.shape[1]
    h = jax.nn.relu(_bdot(x, W_in) + b_in)
    src = edge_index[0]
    dst = edge_index[1]
    msg = jax.ops.segment_sum(h[src], dst, num_segments=N)
    deg = jax.ops.segment_sum(jnp.ones((src.shape[0],), jnp.float32), dst, num_segments=N)
    msg = msg / jnp.maximum(deg, 1.0)[:, None]
    h2 = jax.nn.relu(_bdot(h + msg, W_h) + b_h)
    psum = jax.ops.segment_sum(h2, batch, num_segments=Bn)
    pcnt = jax.ops.segment_sum(jnp.ones((N,), jnp.float32), batch, num_segments=Bn)
    pooled = psum / jnp.maximum(pcnt, 1.0)[:, None]
    z = _bdot(pooled, W_z) + b_z
    r = z
    z_q = jnp.zeros_like(z)
    idx_list = []
    for l in range(codebooks.shape[0]):
        codes = _bdot(codebooks[l], proj[l])
        d2 = jnp.sum(r * r, axis=1, keepdims=True) - 2.0 * _bdot(r, codes.T) + jnp.sum(codes * codes, axis=1)[None, :]
        idx = jnp.argmin(d2, axis=1)
        q = codes[idx]
        z_q = z_q + q
        r = r - q
        idx_list.append(idx)
    indices = jnp.stack(idx_list, axis=1)
    z_q_st = z + jax.lax.stop_gradient(z_q - z)
    nkv = W_kv.shape[1] // d_embed
    kv = (z_q_st @ W_kv + b_kv).reshape(Bn, nkv, d_embed)
    logits = jnp.einsum('md,bkd->bmk', query_emb, kv) / jnp.sqrt(jnp.float32(d_embed))
    att = jax.nn.softmax(logits, axis=-1)
    ctx = jnp.einsum('bmk,bkd->bmd', att, kv)
    recon = jax.nn.relu(ctx @ W_d1 + b_d1) @ W_d2 + b_d2
    mask = (jnp.arange(M)[None, :] < n_vertices[:, None]).astype(jnp.float32)
    diff = recon[:, :, None, :] - gt_vertices[:, None, :, :]
    dist = jnp.sum(diff * diff, axis=-1)
    big = 1e9
    d_rg = dist + (1.0 - mask[:, None, :]) * big
    min_rg = jnp.min(d_rg, axis=2)
    denom = jnp.maximum(jnp.sum(mask, axis=1), 1.0)
    loss1 = jnp.sum(min_rg * mask, axis=1) / denom
    d_gr = dist + (1.0 - mask[:, :, None]) * big
    min_gr = jnp.min(d_gr, axis=1)
    loss2 = jnp.sum(min_gr * mask, axis=1) / denom
    recon_loss = jnp.mean(loss1 + loss2)
    commit_loss = jnp.mean((z - jax.lax.stop_gradient(z_q)) ** 2)
    embed_loss = jnp.mean((jax.lax.stop_gradient(z) - z_q) ** 2)
    total_loss = _combine(recon_loss, commit_loss + embed_loss)
    return (recon, total_loss, recon_loss, commit_loss, embed_loss, indices, z)


# speedup vs baseline: 1.6550x; 1.6042x over previous
import functools

import jax
import jax.numpy as jnp
from jax import lax
from jax.experimental import pallas as pl
from jax.experimental.pallas import tpu as pltpu
from jax.experimental.pallas import tpu_sc as plsc

N_NODES = 50000
N_PAD = 50176
ROWS_PER_TILE = N_PAD // 16
E_RAW = 800000
G = 1024
E_PAD = 819200
E_PER_W = E_PAD // 32
NSLICE = 16
SLICE_W = 16


def _edge_kernel_body(srcv8, dst2, hf, zeros2, zeros1, ones_h,
                      msg_out, deg_out,
                      idx_v, dst_v, rows_v, ones_v, zed_v, acc_s, deg_s, sem):
    c = lax.axis_index("c")
    s = lax.axis_index("s")
    wid = c * 16 + s
    base = wid * E_PER_W
    drow0 = wid * (E_PER_W // 128)

    for p in range(NSLICE):
        pltpu.sync_copy(zeros2, acc_s.at[pl.ds(s * ROWS_PER_TILE, ROWS_PER_TILE)])
        if p == 0:
            pltpu.sync_copy(zeros1, zed_v)
            pltpu.sync_copy(zed_v, deg_s.at[pl.ds(s * ROWS_PER_TILE, ROWS_PER_TILE)])
            pltpu.sync_copy(ones_h, ones_v)
        plsc.subcore_barrier()

        def chunk(j, carry, p=p):
            pltpu.sync_copy(srcv8.at[pl.ds(p * E_PAD + base + j * G, G)], idx_v)
            pltpu.async_copy(hf.at[idx_v], rows_v, sem).wait()
            pltpu.sync_copy(dst2.at[pl.ds(drow0 + j * 8, 8)], dst_v)
            for q in range(8):
                pltpu.sync_copy(rows_v.at[pl.ds(q * 128, 128)],
                                acc_s.at[dst_v.at[q]], add=True)
                if p == 0:
                    pltpu.sync_copy(ones_v, deg_s.at[dst_v.at[q]], add=True)
            return carry

        lax.fori_loop(0, E_PER_W // G, chunk, 0)
        plsc.subcore_barrier()

        obase = (c * NSLICE + p) * N_PAD + s * ROWS_PER_TILE
        pltpu.sync_copy(acc_s.at[pl.ds(s * ROWS_PER_TILE, ROWS_PER_TILE)],
                        msg_out.at[pl.ds(obase, ROWS_PER_TILE)])
        if p == 0:
            pltpu.sync_copy(deg_s.at[pl.ds(s * ROWS_PER_TILE, ROWS_PER_TILE)],
                            zed_v)
            pltpu.sync_copy(zed_v,
                            deg_out.at[pl.ds(c * N_PAD + s * ROWS_PER_TILE,
                                             ROWS_PER_TILE)])


_edge_kernel = functools.partial(
    pl.kernel,
    mesh=plsc.VectorSubcoreMesh(core_axis_name="c", subcore_axis_name="s"),
    compiler_params=pltpu.CompilerParams(use_tc_tiling_on_sc=False),
    out_type=[
        jax.ShapeDtypeStruct((2 * NSLICE * N_PAD, SLICE_W), jnp.float32),
        jax.ShapeDtypeStruct((2 * N_PAD,), jnp.float32),
    ],
    scratch_types=[
        pltpu.VMEM((G,), jnp.int32),
        pltpu.VMEM((8, 128), jnp.int32),
        pltpu.VMEM((G, SLICE_W), jnp.float32),
        pltpu.VMEM((128,), jnp.float32),
        pltpu.VMEM((ROWS_PER_TILE,), jnp.float32),
        pltpu.VMEM_SHARED((N_PAD, SLICE_W), jnp.float32),
        pltpu.VMEM_SHARED((N_PAD,), jnp.float32),
        pltpu.SemaphoreType.DMA,
    ],
)(_edge_kernel_body)


def _edge_message_pass(h, src, dst):
    npad_e = E_PAD - E_RAW
    padi = (50000 + (jnp.arange(npad_e, dtype=jnp.int32) % 128))
    src_p = jnp.concatenate([src, padi])
    dst_p = jnp.concatenate([dst, padi])
    srcv8 = (src_p[None, :]
             + (jnp.arange(NSLICE, dtype=jnp.int32) * N_PAD)[:, None]).reshape(-1)
    dst2 = dst_p.reshape(E_PAD // 128, 128)
    hp = jnp.pad(h, ((0, N_PAD - N_NODES), (0, 0)))
    hf = hp.reshape(N_PAD, NSLICE, SLICE_W).transpose(1, 0, 2).reshape(
        NSLICE * N_PAD, SLICE_W)
    zeros2 = jnp.zeros((ROWS_PER_TILE, SLICE_W), jnp.float32)
    zeros1 = jnp.zeros((ROWS_PER_TILE,), jnp.float32)
    ones_h = jnp.ones((128,), jnp.float32)
    msg_out, deg_out = _edge_kernel(srcv8, dst2, hf, zeros2, zeros1, ones_h)
    msg_t = msg_out.reshape(2, NSLICE, N_PAD, SLICE_W)
    msg_t = msg_t[0] + msg_t[1]
    msg = msg_t.transpose(1, 0, 2).reshape(N_PAD, 256)[:N_NODES]
    deg = deg_out.reshape(2, N_PAD).sum(axis=0)[:N_NODES]
    return msg, deg


def _bdot(a, b):
    return jax.lax.dot(a.astype(jnp.bfloat16), b.astype(jnp.bfloat16),
                       preferred_element_type=jnp.float32)


def kernel(x, edge_index, batch, n_vertices, gt_vertices, W_in, b_in, W_h,
           b_h, W_z, b_z, codebooks, proj, query_emb, W_kv, b_kv, W_d1, b_d1,
           W_d2, b_d2):
    N = x.shape[0]
    Bn = n_vertices.shape[0]
    M = gt_vertices.shape[1]
    d_embed = W_z.shape[1]
    h = jax.nn.relu(_bdot(x, W_in) + b_in)
    src = edge_index[0]
    dst = edge_index[1]
    msg, deg = _edge_message_pass(h, src, dst)
    msg = msg / jnp.maximum(deg, 1.0)[:, None]
    h2 = jax.nn.relu(_bdot(h + msg, W_h) + b_h)
    psum = jax.ops.segment_sum(h2, batch, num_segments=Bn)
    pcnt = jax.ops.segment_sum(jnp.ones((N,), jnp.float32), batch,
                               num_segments=Bn)
    pooled = psum / jnp.maximum(pcnt, 1.0)[:, None]
    z = _bdot(pooled, W_z) + b_z
    r = z
    z_q = jnp.zeros_like(z)
    idx_list = []
    for l in range(codebooks.shape[0]):
        codes = _bdot(codebooks[l], proj[l])
        d2 = (jnp.sum(r * r, axis=1, keepdims=True)
              - 2.0 * _bdot(r, codes.T)
              + jnp.sum(codes * codes, axis=1)[None, :])
        idx = jnp.argmin(d2, axis=1)
        q = codes[idx]
        z_q = z_q + q
        r = r - q
        idx_list.append(idx)
    indices = jnp.stack(idx_list, axis=1)
    z_q_st = z + jax.lax.stop_gradient(z_q - z)
    nkv = W_kv.shape[1] // d_embed
    kv = (z_q_st @ W_kv + b_kv).reshape(Bn, nkv, d_embed)
    logits = jnp.einsum('md,bkd->bmk', query_emb, kv) / jnp.sqrt(
        jnp.float32(d_embed))
    att = jax.nn.softmax(logits, axis=-1)
    ctx = jnp.einsum('bmk,bkd->bmd', att, kv)
    recon = jax.nn.relu(ctx @ W_d1 + b_d1) @ W_d2 + b_d2
    mask = (jnp.arange(M)[None, :] < n_vertices[:, None]).astype(jnp.float32)
    diff = recon[:, :, None, :] - gt_vertices[:, None, :, :]
    dist = jnp.sum(diff * diff, axis=-1)
    big = 1e9
    d_rg = dist + (1.0 - mask[:, None, :]) * big
    min_rg = jnp.min(d_rg, axis=2)
    denom = jnp.maximum(jnp.sum(mask, axis=1), 1.0)
    loss1 = jnp.sum(min_rg * mask, axis=1) / denom
    d_gr = dist + (1.0 - mask[:, :, None]) * big
    min_gr = jnp.min(d_gr, axis=1)
    loss2 = jnp.sum(min_gr * mask, axis=1) / denom
    recon_loss = jnp.mean(loss1 + loss2)
    commit_loss = jnp.mean((z - jax.lax.stop_gradient(z_q)) ** 2)
    embed_loss = jnp.mean((jax.lax.stop_gradient(z) - z_q) ** 2)
    total_loss = recon_loss + commit_loss + embed_loss
    return (recon, total_loss, recon_loss, commit_loss, embed_loss, indices, z)


# speedup vs baseline: 1.7958x; 1.0850x over previous
import functools

import jax
import jax.numpy as jnp
from jax import lax
from jax.experimental import pallas as pl
from jax.experimental.pallas import tpu as pltpu
from jax.experimental.pallas import tpu_sc as plsc

N_NODES = 50000
N_PAD = 50176
ROWS_PER_TILE = N_PAD // 16
E_RAW = 800000
G = 1024
E_PAD = 819200
E_PER_W = E_PAD // 32
NSLICE = 16
SLICE_W = 16


def _edge_kernel_body(srcv8, dst2, hf, zeros2, zeros1, ones_h,
                      msg_out, deg_out,
                      idx_v, dst_v, rows_v, ones_v, zed_v, acc_s, deg_s, sem):
    c = lax.axis_index("c")
    s = lax.axis_index("s")
    wid = c * 16 + s
    base = wid * E_PER_W
    drow0 = wid * (E_PER_W // 128)

    for p in range(NSLICE):
        pltpu.sync_copy(zeros2, acc_s.at[pl.ds(s * ROWS_PER_TILE, ROWS_PER_TILE)])
        if p == 0:
            pltpu.sync_copy(zeros1, zed_v)
            pltpu.sync_copy(zed_v, deg_s.at[pl.ds(s * ROWS_PER_TILE, ROWS_PER_TILE)])
            pltpu.sync_copy(ones_h, ones_v)
        plsc.subcore_barrier()

        def chunk(j, carry, p=p):
            pltpu.sync_copy(srcv8.at[pl.ds(p * E_PAD + base + j * G, G)], idx_v)
            pltpu.async_copy(hf.at[idx_v], rows_v, sem).wait()
            pltpu.sync_copy(dst2.at[pl.ds(drow0 + j * 8, 8)], dst_v)
            for q in range(8):
                pltpu.sync_copy(rows_v.at[pl.ds(q * 128, 128)],
                                acc_s.at[dst_v.at[q]], add=True)
                if p == 0:
                    pltpu.sync_copy(ones_v, deg_s.at[dst_v.at[q]], add=True)
            return carry

        lax.fori_loop(0, E_PER_W // G, chunk, 0)
        plsc.subcore_barrier()

        obase = (c * NSLICE + p) * N_PAD + s * ROWS_PER_TILE
        pltpu.sync_copy(acc_s.at[pl.ds(s * ROWS_PER_TILE, ROWS_PER_TILE)],
                        msg_out.at[pl.ds(obase, ROWS_PER_TILE)])
        if p == 0:
            pltpu.sync_copy(deg_s.at[pl.ds(s * ROWS_PER_TILE, ROWS_PER_TILE)],
                            zed_v)
            pltpu.sync_copy(zed_v,
                            deg_out.at[pl.ds(c * N_PAD + s * ROWS_PER_TILE,
                                             ROWS_PER_TILE)])


_edge_kernel = functools.partial(
    pl.kernel,
    mesh=plsc.VectorSubcoreMesh(core_axis_name="c", subcore_axis_name="s"),
    compiler_params=pltpu.CompilerParams(use_tc_tiling_on_sc=False),
    out_type=[
        jax.ShapeDtypeStruct((2 * NSLICE * N_PAD, SLICE_W), jnp.float32),
        jax.ShapeDtypeStruct((2 * N_PAD,), jnp.float32),
    ],
    scratch_types=[
        pltpu.VMEM((G,), jnp.int32),
        pltpu.VMEM((8, 128), jnp.int32),
        pltpu.VMEM((G, SLICE_W), jnp.float32),
        pltpu.VMEM((128,), jnp.float32),
        pltpu.VMEM((ROWS_PER_TILE,), jnp.float32),
        pltpu.VMEM_SHARED((N_PAD, SLICE_W), jnp.float32),
        pltpu.VMEM_SHARED((N_PAD,), jnp.float32),
        pltpu.SemaphoreType.DMA,
    ],
)(_edge_kernel_body)


def _edge_message_pass(hp, src, dst):
    npad_e = E_PAD - E_RAW
    padi = (50000 + (jnp.arange(npad_e, dtype=jnp.int32) % 128))
    src_p = jnp.concatenate([src, padi])
    dst_p = jnp.concatenate([dst, padi])
    srcv8 = (src_p[None, :]
             + (jnp.arange(NSLICE, dtype=jnp.int32) * N_PAD)[:, None]).reshape(-1)
    dst2 = dst_p.reshape(E_PAD // 128, 128)
    hf = hp.reshape(N_PAD, NSLICE, SLICE_W).transpose(1, 0, 2).reshape(
        NSLICE * N_PAD, SLICE_W)
    zeros2 = jnp.zeros((ROWS_PER_TILE, SLICE_W), jnp.float32)
    zeros1 = jnp.zeros((ROWS_PER_TILE,), jnp.float32)
    ones_h = jnp.ones((128,), jnp.float32)
    msg_out, deg_out = _edge_kernel(srcv8, dst2, hf, zeros2, zeros1, ones_h)
    msg_t = msg_out.reshape(2, NSLICE, N_PAD, SLICE_W)
    msg_t = msg_t[0] + msg_t[1]
    msg = msg_t.transpose(1, 0, 2).reshape(N_PAD, 256)
    deg = deg_out.reshape(2, N_PAD).sum(axis=0)
    return msg, deg


def _bdot(a, b):
    return jax.lax.dot(a.astype(jnp.bfloat16), b.astype(jnp.bfloat16),
                       preferred_element_type=jnp.float32)


NB = N_PAD // 512


def _pool_kernel_body(h_ref, msg_ref, deg_ref, bids_ref, wh_ref, bh_ref,
                      psum_ref, pcnt_ref):
    i = pl.program_id(0)
    h = h_ref[...]
    m = msg_ref[...] / jnp.maximum(deg_ref[...], 1.0)
    pre = jax.lax.dot((h + m).astype(jnp.bfloat16),
                      wh_ref[...].astype(jnp.bfloat16),
                      preferred_element_type=jnp.float32) + bh_ref[...]
    h2 = jnp.maximum(pre, 0.0)
    bids = bids_ref[...].reshape(1, 512)
    pid_col = jax.lax.broadcasted_iota(jnp.int32, (512, 512), 0)
    oh = (pid_col == jnp.broadcast_to(bids, (512, 512))).astype(jnp.float32)
    ps = jax.lax.dot(oh, h2, preferred_element_type=jnp.float32,
                     precision=jax.lax.Precision.HIGHEST)
    cnt = jnp.sum(oh, axis=1, keepdims=True)

    @pl.when(i == 0)
    def _init():
        psum_ref[...] = jnp.zeros_like(psum_ref)
        pcnt_ref[...] = jnp.zeros_like(pcnt_ref)

    psum_ref[...] += ps
    pcnt_ref[...] += cnt


def _pooled_encode(h_pad, msg_t2, deg2, batch3, W_h, b_h):
    psum, pcnt = pl.pallas_call(
        _pool_kernel_body,
        grid=(NB,),
        in_specs=[
            pl.BlockSpec((512, 256), lambda i: (i, 0)),
            pl.BlockSpec((512, 256), lambda i: (i, 0)),
            pl.BlockSpec((512, 1), lambda i: (i, 0)),
            pl.BlockSpec((1, 1, 512), lambda i: (i, 0, 0)),
            pl.BlockSpec((256, 256), lambda i: (0, 0)),
            pl.BlockSpec((1, 256), lambda i: (0, 0)),
        ],
        out_specs=[
            pl.BlockSpec((512, 256), lambda i: (0, 0)),
            pl.BlockSpec((512, 1), lambda i: (0, 0)),
        ],
        out_shape=[
            jax.ShapeDtypeStruct((512, 256), jnp.float32),
            jax.ShapeDtypeStruct((512, 1), jnp.float32),
        ],
    )(h_pad, msg_t2, deg2, batch3, W_h, b_h.reshape(1, 256))
    return psum, pcnt


def kernel(x, edge_index, batch, n_vertices, gt_vertices, W_in, b_in, W_h,
           b_h, W_z, b_z, codebooks, proj, query_emb, W_kv, b_kv, W_d1, b_d1,
           W_d2, b_d2):
    N = x.shape[0]
    Bn = n_vertices.shape[0]
    M = gt_vertices.shape[1]
    d_embed = W_z.shape[1]
    h = jax.nn.relu(_bdot(x, W_in) + b_in)
    h_pad = jnp.pad(h, ((0, N_PAD - N_NODES), (0, 0)))
    src = edge_index[0]
    dst = edge_index[1]
    msg_p, deg_p = _edge_message_pass(h_pad, src, dst)
    batch3 = jnp.pad(batch, (0, N_PAD - N_NODES),
                     constant_values=Bn).reshape(NB, 1, 512)
    psum, pcnt = _pooled_encode(h_pad, msg_p, deg_p[:, None], batch3,
                                W_h, b_h)
    pooled = psum / jnp.maximum(pcnt, 1.0)
    z = _bdot(pooled, W_z) + b_z
    r = z
    z_q = jnp.zeros_like(z)
    idx_list = []
    for l in range(codebooks.shape[0]):
        codes = _bdot(codebooks[l], proj[l])
        d2 = (jnp.sum(r * r, axis=1, keepdims=True)
              - 2.0 * _bdot(r, codes.T)
              + jnp.sum(codes * codes, axis=1)[None, :])
        idx = jnp.argmin(d2, axis=1)
        q = codes[idx]
        z_q = z_q + q
        r = r - q
        idx_list.append(idx)
    indices = jnp.stack(idx_list, axis=1)
    z_q_st = z + jax.lax.stop_gradient(z_q - z)
    nkv = W_kv.shape[1] // d_embed
    kv = (z_q_st @ W_kv + b_kv).reshape(Bn, nkv, d_embed)
    logits = jnp.einsum('md,bkd->bmk', query_emb, kv) / jnp.sqrt(
        jnp.float32(d_embed))
    att = jax.nn.softmax(logits, axis=-1)
    ctx = jnp.einsum('bmk,bkd->bmd', att, kv)
    recon = jax.nn.relu(ctx @ W_d1 + b_d1) @ W_d2 + b_d2
    mask = (jnp.arange(M)[None, :] < n_vertices[:, None]).astype(jnp.float32)
    diff = recon[:, :, None, :] - gt_vertices[:, None, :, :]
    dist = jnp.sum(diff * diff, axis=-1)
    big = 1e9
    d_rg = dist + (1.0 - mask[:, None, :]) * big
    min_rg = jnp.min(d_rg, axis=2)
    denom = jnp.maximum(jnp.sum(mask, axis=1), 1.0)
    loss1 = jnp.sum(min_rg * mask, axis=1) / denom
    d_gr = dist + (1.0 - mask[:, :, None]) * big
    min_gr = jnp.min(d_gr, axis=1)
    loss2 = jnp.sum(min_gr * mask, axis=1) / denom
    recon_loss = jnp.mean(loss1 + loss2)
    commit_loss = jnp.mean((z - jax.lax.stop_gradient(z_q)) ** 2)
    embed_loss = jnp.mean((jax.lax.stop_gradient(z) - z_q) ** 2)
    total_loss = recon_loss + commit_loss + embed_loss
    return (recon, total_loss, recon_loss, commit_loss, embed_loss, indices, z)
